# baseline (speedup 1.0000x reference)
import functools

import jax
import jax.numpy as jnp
from jax import lax
from jax.experimental import pallas as pl
from jax.experimental.pallas import tpu as pltpu
from jax.experimental.pallas import tpu_sc as plsc

N = 10000
D_IN = 256
D_OUT = 256
H = 128
E = 160000
NS = 16
NP = NS * 640
RPT = 640

K_AGG = 64
NCH_AGG = 160
W_AGG = 20
NWIN = NCH_AGG // W_AGG
EPW = NCH_AGG * K_AGG

K_DEG = 128
NCH_DEG = 80
EP = NS * K_DEG * NCH_DEG


def _mesh():
    return plsc.VectorSubcoreMesh(core_axis_name="c", subcore_axis_name="s")


def _copy_node_rows(src, dst, s):
    pltpu.sync_copy(src.at[pl.ds(s * RPT, RPT)], dst.at[pl.ds(s * RPT, RPT)])


def _deg_body(dst_s, dst_t, ones_h, zeros_h, deg_s, deg_t, idx_v, ones_v, deg_sp):
    c = lax.axis_index("c")
    s = lax.axis_index("s")
    pltpu.sync_copy(ones_h, ones_v)
    pltpu.sync_copy(zeros_h, deg_sp.at[pl.ds(s * RPT, RPT)])

    @pl.when(c == 0)
    def _():
        pltpu.sync_copy(dst_s.at[s], idx_v)

    @pl.when(c == 1)
    def _():
        pltpu.sync_copy(dst_t.at[s], idx_v)

    plsc.subcore_barrier()

    def body(i, carry):
        pltpu.sync_copy(ones_v, deg_sp.at[idx_v.at[i]], add=True)
        return carry

    lax.fori_loop(0, NCH_DEG, body, 0)
    plsc.subcore_barrier()

    @pl.when(c == 0)
    def _():
        _copy_node_rows(deg_sp, deg_s, s)

    @pl.when(c == 1)
    def _():
        _copy_node_rows(deg_sp, deg_t, s)


@functools.cache
def _deg_kernel():
    return pl.kernel(
        _deg_body,
        out_type=[jax.ShapeDtypeStruct((NP,), jnp.float32)] * 2,
        mesh=_mesh(),
        scratch_types=[
            pltpu.VMEM((NCH_DEG, K_DEG), jnp.int32),
            pltpu.VMEM((K_DEG,), jnp.float32),
            pltpu.VMEM_SHARED((NP,), jnp.float32),
        ],
    )


_BM = 1000


def _scale_body(x_ref, w_ref, dgs_ref, dgt_ref, ys0, ys1, yt0, yt1):
    xw = jnp.dot(x_ref[...].astype(jnp.bfloat16),
                 w_ref[...].astype(jnp.bfloat16),
                 preferred_element_type=jnp.float32)
    dis_s = lax.rsqrt(dgs_ref[...] + 1.0)
    dis_t = lax.rsqrt(dgt_ref[...] + 1.0)
    ys = xw[:, :D_OUT] * dis_s
    yt = xw[:, D_OUT:] * dis_t
    ys0[...] = ys[:, :H]
    ys1[...] = ys[:, H:]
    yt0[...] = yt[:, :H]
    yt1[...] = yt[:, H:]


def _scale_matmul(x, w_cat, deg_s, deg_t):
    return pl.pallas_call(
        _scale_body,
        grid=(N // _BM,),
        in_specs=[
            pl.BlockSpec((_BM, D_IN), lambda i: (i, 0)),
            pl.BlockSpec((D_IN, 2 * D_OUT), lambda i: (0, 0)),
            pl.BlockSpec((_BM, 1), lambda i: (i, 0)),
            pl.BlockSpec((_BM, 1), lambda i: (i, 0)),
        ],
        out_specs=[pl.BlockSpec((_BM, H), lambda i: (i, 0))] * 4,
        out_shape=[jax.ShapeDtypeStruct((NP, H), jnp.float32)] * 4,
    )(x, w_cat, deg_s, deg_t)


_NBUF = 4


def _agg_body(y_s0, y_s1, y_t0, y_t1, src_e, dst_e,
              o_s0, o_s1, o_t0, o_t1,
              src_v, win_a, win_b, r0, r1, r2, r3, sem_g, sem_s, sem_w, agg_sp):
    c = lax.axis_index("c")
    s = lax.axis_index("s")
    rows = [r0, r1, r2, r3]
    wins = [win_a, win_b]

    def gather_chunk(y_ref, i, buf):
        pltpu.async_copy(y_ref.at[src_v.at[pl.ds(i * K_AGG, K_AGG)]],
                         rows[buf], sem_g)

    def wait_gather(y_ref, i, buf):
        pltpu.make_async_copy(y_ref.at[src_v.at[pl.ds(i * K_AGG, K_AGG)]],
                              rows[buf], sem_g).wait()

    def wait_scatter(buf, win, r):
        pltpu.make_async_copy(rows[buf], agg_sp.at[win.at[r]], sem_s).wait()

    def run_graph(g, y_ref, o_ref):
        base = (g * NS + s) * NWIN
        pltpu.sync_copy(src_e.at[g * NS + s], src_v)
        pltpu.sync_copy(dst_e.at[base], win_a)
        pltpu.sync_copy(dst_e.at[base + 1], win_b)
        _copy_node_rows(y_ref, agg_sp, s)
        plsc.subcore_barrier()

        for b in range(2):
            gather_chunk(y_ref, b, b)

        def do_window(w, win, other, wait_stage, drain_prev, stage_other,
                      stage_idx):
            if wait_stage is not None:
                @pl.when(wait_stage)
                def _():
                    pltpu.make_async_copy(dst_e.at[base], win, sem_w).wait()

            for r in range(W_AGG):
                i = w * W_AGG + r
                b = r % _NBUF
                wait_gather(y_ref, i, b)
                pltpu.async_copy(rows[b], agg_sp.at[win.at[r]], sem_s,
                                 add=True)
                if r >= 2:
                    wait_scatter((r - 2) % _NBUF, win, r - 2)
                elif drain_prev is True:
                    wait_scatter((r + 2) % _NBUF, other, r - 2 + W_AGG)
                else:
                    @pl.when(drain_prev)
                    def _():
                        wait_scatter((r + 2) % _NBUF, other, r - 2 + W_AGG)

                if r == 1 and stage_other is not None:
                    @pl.when(stage_other)
                    def _():
                        pltpu.async_copy(dst_e.at[stage_idx], other, sem_w)

                @pl.when(i + 2 < NCH_AGG)
                def _():
                    gather_chunk(y_ref, i + 2, (r + 2) % _NBUF)

        def pair(k, carry):
            w0 = 2 * k
            do_window(w0, win_a, win_b,
                      wait_stage=k > 0, drain_prev=k > 0,
                      stage_other=k > 0, stage_idx=base + w0 + 1)
            do_window(w0 + 1, win_b, win_a,
                      wait_stage=k > 0, drain_prev=True,
                      stage_other=w0 + 2 < NWIN, stage_idx=base + w0 + 2)
            return carry

        lax.fori_loop(0, NWIN // 2, pair, 0)
        wait_scatter((W_AGG - 2) % _NBUF, win_b, W_AGG - 2)
        wait_scatter((W_AGG - 1) % _NBUF, win_b, W_AGG - 1)
        plsc.subcore_barrier()
        _copy_node_rows(agg_sp, o_ref, s)
        plsc.subcore_barrier()

    @pl.when(c == 0)
    def _():
        run_graph(0, y_s0, o_s0)
        run_graph(1, y_t0, o_t0)

    @pl.when(c == 1)
    def _():
        run_graph(0, y_s1, o_s1)
        run_graph(1, y_t1, o_t1)


@functools.cache
def _agg_kernel():
    return pl.kernel(
        _agg_body,
        out_type=[jax.ShapeDtypeStruct((NP, H), jnp.float32)] * 4,
        mesh=_mesh(),
        scratch_types=[
            pltpu.VMEM((EPW,), jnp.int32),
            pltpu.VMEM((W_AGG, K_AGG), jnp.int32),
            pltpu.VMEM((W_AGG, K_AGG), jnp.int32),
            pltpu.VMEM((K_AGG, H), jnp.float32),
            pltpu.VMEM((K_AGG, H), jnp.float32),
            pltpu.VMEM((K_AGG, H), jnp.float32),
            pltpu.VMEM((K_AGG, H), jnp.float32),
            pltpu.SemaphoreType.DMA,
            pltpu.SemaphoreType.DMA,
            pltpu.SemaphoreType.DMA,
            pltpu.VMEM_SHARED((NP, H), jnp.float32),
        ],
    )


def _fuse_body(as0, as1, at0, at1, dgs_ref, dgt_ref, wf_ref, bs_ref, bt_ref,
               bf_ref, out_ref):
    dis_s = lax.rsqrt(dgs_ref[...] + 1.0)
    dis_t = lax.rsqrt(dgt_ref[...] + 1.0)
    hs_l = jnp.maximum(as0[...] * dis_s + bs_ref[:, :H], 0.0).astype(jnp.bfloat16)
    hs_r = jnp.maximum(as1[...] * dis_s + bs_ref[:, H:], 0.0).astype(jnp.bfloat16)
    ht_l = jnp.maximum(at0[...] * dis_t + bt_ref[:, :H], 0.0).astype(jnp.bfloat16)
    ht_r = jnp.maximum(at1[...] * dis_t + bt_ref[:, H:], 0.0).astype(jnp.bfloat16)
    wf = wf_ref[...].astype(jnp.bfloat16)
    acc = bf_ref[...]
    acc = acc + jnp.dot(hs_l, wf[0:H], preferred_element_type=jnp.float32)
    acc = acc + jnp.dot(hs_r, wf[H:2 * H], preferred_element_type=jnp.float32)
    acc = acc + jnp.dot(ht_l, wf[2 * H:3 * H], preferred_element_type=jnp.float32)
    acc = acc + jnp.dot(ht_r, wf[3 * H:4 * H], preferred_element_type=jnp.float32)
    out_ref[...] = jnp.maximum(acc, 0.0)


def _fuse(as0, as1, at0, at1, deg_s, deg_t, w_fuse, b_spa, b_tra, b_fuse):
    return pl.pallas_call(
        _fuse_body,
        grid=(N // _BM,),
        in_specs=[
            pl.BlockSpec((_BM, H), lambda i: (i, 0)),
            pl.BlockSpec((_BM, H), lambda i: (i, 0)),
            pl.BlockSpec((_BM, H), lambda i: (i, 0)),
            pl.BlockSpec((_BM, H), lambda i: (i, 0)),
            pl.BlockSpec((_BM, 1), lambda i: (i, 0)),
            pl.BlockSpec((_BM, 1), lambda i: (i, 0)),
            pl.BlockSpec((2 * D_OUT, D_OUT), lambda i: (0, 0)),
            pl.BlockSpec((1, D_OUT), lambda i: (0, 0)),
            pl.BlockSpec((1, D_OUT), lambda i: (0, 0)),
            pl.BlockSpec((1, D_OUT), lambda i: (0, 0)),
        ],
        out_specs=pl.BlockSpec((_BM, D_OUT), lambda i: (i, 0)),
        out_shape=jax.ShapeDtypeStruct((N, D_OUT), jnp.float32),
    )(as0, as1, at0, at1, deg_s, deg_t, w_fuse, b_spa, b_tra, b_fuse)


def kernel(x, sp_ei, tr_ei, W_spa, b_spa, W_tra, b_tra, W_fuse, b_fuse):
    sp_ei = sp_ei.astype(jnp.int32)
    tr_ei = tr_ei.astype(jnp.int32)

    pad = N + jnp.arange(EP - E, dtype=jnp.int32) % (NP - N)
    dst_s_p = jnp.concatenate([sp_ei[1], pad]).reshape(NS, NCH_DEG, K_DEG)
    dst_t_p = jnp.concatenate([tr_ei[1], pad]).reshape(NS, NCH_DEG, K_DEG)
    ones_h = jnp.ones((K_DEG,), jnp.float32)
    zeros_h = jnp.zeros((RPT,), jnp.float32)
    deg_s, deg_t = _deg_kernel()(dst_s_p, dst_t_p, ones_h, zeros_h)
    deg_s = deg_s.reshape(NP, 1)
    deg_t = deg_t.reshape(NP, 1)

    w_cat = jnp.concatenate([W_spa, W_tra], axis=1)
    ys0, ys1, yt0, yt1 = _scale_matmul(x, w_cat, deg_s, deg_t)

    pad_a = (N + jnp.arange(EPW - E // NS, dtype=jnp.int32) % (NP - N))
    pad_a = jnp.broadcast_to(pad_a, (2 * NS, EPW - E // NS))

    def _pad_edges(a):
        return jnp.concatenate([a.reshape(2 * NS, E // NS), pad_a], axis=1)

    src_e = _pad_edges(jnp.stack([sp_ei[0], tr_ei[0]]))
    dst_e = _pad_edges(jnp.stack([sp_ei[1], tr_ei[1]]))
    dst_e = dst_e.reshape(2 * NS * NWIN, W_AGG, K_AGG)
    os0, os1, ot0, ot1 = _agg_kernel()(ys0, ys1, yt0, yt1, src_e, dst_e)

    return _fuse(os0, os1, ot0, ot1, deg_s, deg_t, W_fuse,
                 b_spa.reshape(1, D_OUT), b_tra.reshape(1, D_OUT),
                 b_fuse.reshape(1, D_OUT))


# speedup vs baseline: 1.0829x; 1.0829x over previous
import functools

import jax
import jax.numpy as jnp
from jax import lax
from jax.experimental import pallas as pl
from jax.experimental.pallas import tpu as pltpu
from jax.experimental.pallas import tpu_sc as plsc

N = 10000
D_IN = 256
D_OUT = 256
H = 128
E = 160000
NS = 16
NP = NS * 640
RPT = 640

K_AGG = 80
NCH_AGG = 126
W_AGG = 42
NWIN = NCH_AGG // W_AGG
EPW = NCH_AGG * K_AGG

K_DEG = 128
NCH_DEG = 80
EP = NS * K_DEG * NCH_DEG


def _mesh():
    return plsc.VectorSubcoreMesh(core_axis_name="c", subcore_axis_name="s")


def _copy_node_rows(src, dst, s):
    pltpu.sync_copy(src.at[pl.ds(s * RPT, RPT)], dst.at[pl.ds(s * RPT, RPT)])


def _deg_body(dst_s, dst_t, ones_h, zeros_h, deg_s, deg_t, idx_v, ones_v, deg_sp):
    c = lax.axis_index("c")
    s = lax.axis_index("s")
    pltpu.sync_copy(ones_h, ones_v)
    pltpu.sync_copy(zeros_h, deg_sp.at[pl.ds(s * RPT, RPT)])

    @pl.when(c == 0)
    def _():
        pltpu.sync_copy(dst_s.at[s], idx_v)

    @pl.when(c == 1)
    def _():
        pltpu.sync_copy(dst_t.at[s], idx_v)

    plsc.subcore_barrier()

    def body(i, carry):
        pltpu.sync_copy(ones_v, deg_sp.at[idx_v.at[i]], add=True)
        return carry

    lax.fori_loop(0, NCH_DEG, body, 0)
    plsc.subcore_barrier()

    @pl.when(c == 0)
    def _():
        _copy_node_rows(deg_sp, deg_s, s)

    @pl.when(c == 1)
    def _():
        _copy_node_rows(deg_sp, deg_t, s)


@functools.cache
def _deg_kernel():
    return pl.kernel(
        _deg_body,
        out_type=[jax.ShapeDtypeStruct((NP,), jnp.float32)] * 2,
        mesh=_mesh(),
        scratch_types=[
            pltpu.VMEM((NCH_DEG, K_DEG), jnp.int32),
            pltpu.VMEM((K_DEG,), jnp.float32),
            pltpu.VMEM_SHARED((NP,), jnp.float32),
        ],
    )


_BM = 1000


def _scale_body(x_ref, w_ref, dgs_ref, dgt_ref, ys0, ys1, yt0, yt1):
    xw = jnp.dot(x_ref[...].astype(jnp.bfloat16),
                 w_ref[...].astype(jnp.bfloat16),
                 preferred_element_type=jnp.float32)
    dis_s = lax.rsqrt(dgs_ref[...] + 1.0)
    dis_t = lax.rsqrt(dgt_ref[...] + 1.0)
    ys = xw[:, :D_OUT] * dis_s
    yt = xw[:, D_OUT:] * dis_t
    ys0[...] = ys[:, :H]
    ys1[...] = ys[:, H:]
    yt0[...] = yt[:, :H]
    yt1[...] = yt[:, H:]


def _scale_matmul(x, w_cat, deg_s, deg_t):
    return pl.pallas_call(
        _scale_body,
        grid=(N // _BM,),
        in_specs=[
            pl.BlockSpec((_BM, D_IN), lambda i: (i, 0)),
            pl.BlockSpec((D_IN, 2 * D_OUT), lambda i: (0, 0)),
            pl.BlockSpec((_BM, 1), lambda i: (i, 0)),
            pl.BlockSpec((_BM, 1), lambda i: (i, 0)),
        ],
        out_specs=[pl.BlockSpec((_BM, H), lambda i: (i, 0))] * 4,
        out_shape=[jax.ShapeDtypeStruct((NP, H), jnp.float32)] * 4,
    )(x, w_cat, deg_s, deg_t)


_NBUF = 3


def _agg_body(y_s0, y_s1, y_t0, y_t1, src_e, dst_e,
              o_s0, o_s1, o_t0, o_t1,
              src_v, win_v, r0, r1, r2, sem_g, sem_s, agg_sp):
    c = lax.axis_index("c")
    s = lax.axis_index("s")
    rows = [r0, r1, r2]

    def gather_chunk(y_ref, i, buf):
        pltpu.async_copy(y_ref.at[src_v.at[pl.ds(i * K_AGG, K_AGG)]],
                         rows[buf], sem_g)

    def wait_gather(y_ref, i, buf):
        pltpu.make_async_copy(y_ref.at[src_v.at[pl.ds(i * K_AGG, K_AGG)]],
                              rows[buf], sem_g).wait()

    def wait_scatter(buf, r):
        pltpu.make_async_copy(rows[buf], agg_sp.at[win_v.at[r]], sem_s).wait()

    def run_graph(g, y_ref, o_ref):
        pltpu.sync_copy(src_e.at[g * NS + s], src_v)
        _copy_node_rows(y_ref, agg_sp, s)
        plsc.subcore_barrier()

        for b in range(_NBUF - 1):
            gather_chunk(y_ref, b, b)

        def window(w, carry):
            @pl.when(w > 0)
            def _():
                wait_scatter((W_AGG - 1) % _NBUF, W_AGG - 1)

            pltpu.sync_copy(dst_e.at[(g * NS + s) * NWIN + w], win_v)
            for r in range(W_AGG):
                i = w * W_AGG + r
                b = r % _NBUF
                wait_gather(y_ref, i, b)
                pltpu.async_copy(rows[b], agg_sp.at[win_v.at[r]], sem_s,
                                 add=True)
                if r >= 1:
                    wait_scatter((r - 1) % _NBUF, r - 1)

                @pl.when(i + _NBUF - 1 < NCH_AGG)
                def _():
                    gather_chunk(y_ref, i + _NBUF - 1, (r + 2) % _NBUF)
            return carry

        lax.fori_loop(0, NWIN, window, 0)
        wait_scatter((W_AGG - 1) % _NBUF, W_AGG - 1)
        plsc.subcore_barrier()
        _copy_node_rows(agg_sp, o_ref, s)
        plsc.subcore_barrier()

    @pl.when(c == 0)
    def _():
        run_graph(0, y_s0, o_s0)
        run_graph(1, y_t0, o_t0)

    @pl.when(c == 1)
    def _():
        run_graph(0, y_s1, o_s1)
        run_graph(1, y_t1, o_t1)


@functools.cache
def _agg_kernel():
    return pl.kernel(
        _agg_body,
        out_type=[jax.ShapeDtypeStruct((NP, H), jnp.float32)] * 4,
        mesh=_mesh(),
        scratch_types=[
            pltpu.VMEM((EPW,), jnp.int32),
            pltpu.VMEM((W_AGG, K_AGG), jnp.int32),
            pltpu.VMEM((K_AGG, H), jnp.float32),
            pltpu.VMEM((K_AGG, H), jnp.float32),
            pltpu.VMEM((K_AGG, H), jnp.float32),
            pltpu.SemaphoreType.DMA,
            pltpu.SemaphoreType.DMA,
            pltpu.VMEM_SHARED((NP, H), jnp.float32),
        ],
    )


def _fuse_body(as0, as1, at0, at1, dgs_ref, dgt_ref, wf_ref, bs_ref, bt_ref,
               bf_ref, out_ref):
    dis_s = lax.rsqrt(dgs_ref[...] + 1.0)
    dis_t = lax.rsqrt(dgt_ref[...] + 1.0)
    hs_l = jnp.maximum(as0[...] * dis_s + bs_ref[:, :H], 0.0).astype(jnp.bfloat16)
    hs_r = jnp.maximum(as1[...] * dis_s + bs_ref[:, H:], 0.0).astype(jnp.bfloat16)
    ht_l = jnp.maximum(at0[...] * dis_t + bt_ref[:, :H], 0.0).astype(jnp.bfloat16)
    ht_r = jnp.maximum(at1[...] * dis_t + bt_ref[:, H:], 0.0).astype(jnp.bfloat16)
    wf = wf_ref[...].astype(jnp.bfloat16)
    acc = bf_ref[...]
    acc = acc + jnp.dot(hs_l, wf[0:H], preferred_element_type=jnp.float32)
    acc = acc + jnp.dot(hs_r, wf[H:2 * H], preferred_element_type=jnp.float32)
    acc = acc + jnp.dot(ht_l, wf[2 * H:3 * H], preferred_element_type=jnp.float32)
    acc = acc + jnp.dot(ht_r, wf[3 * H:4 * H], preferred_element_type=jnp.float32)
    out_ref[...] = jnp.maximum(acc, 0.0)


def _fuse(as0, as1, at0, at1, deg_s, deg_t, w_fuse, b_spa, b_tra, b_fuse):
    return pl.pallas_call(
        _fuse_body,
        grid=(N // _BM,),
        in_specs=[
            pl.BlockSpec((_BM, H), lambda i: (i, 0)),
            pl.BlockSpec((_BM, H), lambda i: (i, 0)),
            pl.BlockSpec((_BM, H), lambda i: (i, 0)),
            pl.BlockSpec((_BM, H), lambda i: (i, 0)),
            pl.BlockSpec((_BM, 1), lambda i: (i, 0)),
            pl.BlockSpec((_BM, 1), lambda i: (i, 0)),
            pl.BlockSpec((2 * D_OUT, D_OUT), lambda i: (0, 0)),
            pl.BlockSpec((1, D_OUT), lambda i: (0, 0)),
            pl.BlockSpec((1, D_OUT), lambda i: (0, 0)),
            pl.BlockSpec((1, D_OUT), lambda i: (0, 0)),
        ],
        out_specs=pl.BlockSpec((_BM, D_OUT), lambda i: (i, 0)),
        out_shape=jax.ShapeDtypeStruct((N, D_OUT), jnp.float32),
    )(as0, as1, at0, at1, deg_s, deg_t, w_fuse, b_spa, b_tra, b_fuse)


def kernel(x, sp_ei, tr_ei, W_spa, b_spa, W_tra, b_tra, W_fuse, b_fuse):
    sp_ei = sp_ei.astype(jnp.int32)
    tr_ei = tr_ei.astype(jnp.int32)

    pad = N + jnp.arange(EP - E, dtype=jnp.int32) % (NP - N)
    dst_s_p = jnp.concatenate([sp_ei[1], pad]).reshape(NS, NCH_DEG, K_DEG)
    dst_t_p = jnp.concatenate([tr_ei[1], pad]).reshape(NS, NCH_DEG, K_DEG)
    ones_h = jnp.ones((K_DEG,), jnp.float32)
    zeros_h = jnp.zeros((RPT,), jnp.float32)
    deg_s, deg_t = _deg_kernel()(dst_s_p, dst_t_p, ones_h, zeros_h)
    deg_s = deg_s.reshape(NP, 1)
    deg_t = deg_t.reshape(NP, 1)

    w_cat = jnp.concatenate([W_spa, W_tra], axis=1)
    ys0, ys1, yt0, yt1 = _scale_matmul(x, w_cat, deg_s, deg_t)

    pad_a = (N + jnp.arange(EPW - E // NS, dtype=jnp.int32) % (NP - N))
    pad_a = jnp.broadcast_to(pad_a, (2 * NS, EPW - E // NS))

    def _pad_edges(a):
        return jnp.concatenate([a.reshape(2 * NS, E // NS), pad_a], axis=1)

    src_e = _pad_edges(jnp.stack([sp_ei[0], tr_ei[0]]))
    dst_e = _pad_edges(jnp.stack([sp_ei[1], tr_ei[1]]))
    dst_e = dst_e.reshape(2 * NS * NWIN, W_AGG, K_AGG)
    os0, os1, ot0, ot1 = _agg_kernel()(ys0, ys1, yt0, yt1, src_e, dst_e)

    return _fuse(os0, os1, ot0, ot1, deg_s, deg_t, W_fuse,
                 b_spa.reshape(1, D_OUT), b_tra.reshape(1, D_OUT),
                 b_fuse.reshape(1, D_OUT))


# speedup vs baseline: 1.0959x; 1.0120x over previous
import functools

import jax
import jax.numpy as jnp
from jax import lax
from jax.experimental import pallas as pl
from jax.experimental.pallas import tpu as pltpu
from jax.experimental.pallas import tpu_sc as plsc

N = 10000
D_IN = 256
D_OUT = 256
H = 128
E = 160000
NS = 16
NP = NS * 640
RPT = 640

K_AGG = 80
NCH_AGG = 126
W_AGG = 42
NWIN = NCH_AGG // W_AGG
EPW = NCH_AGG * K_AGG

K_DEG = 128
NCH_DEG = 80
EP = NS * K_DEG * NCH_DEG


def _mesh():
    return plsc.VectorSubcoreMesh(core_axis_name="c", subcore_axis_name="s")


def _copy_node_rows(src, dst, s):
    pltpu.sync_copy(src.at[pl.ds(s * RPT, RPT)], dst.at[pl.ds(s * RPT, RPT)])


def _deg_body(dst_s, dst_t, ones_h, zeros_h, deg_s, deg_t, idx_v, ones_v,
              sem_d, deg_sp):
    c = lax.axis_index("c")
    s = lax.axis_index("s")
    pltpu.sync_copy(ones_h, ones_v)
    pltpu.sync_copy(zeros_h, deg_sp.at[pl.ds(s * RPT, RPT)])

    @pl.when(c == 0)
    def _():
        pltpu.sync_copy(dst_s.at[s], idx_v)

    @pl.when(c == 1)
    def _():
        pltpu.sync_copy(dst_t.at[s], idx_v)

    plsc.subcore_barrier()

    def body(j, carry):
        for q in range(8):
            pltpu.async_copy(ones_v, deg_sp.at[idx_v.at[8 * j + q]], sem_d,
                             add=True)
        for q in range(8):
            pltpu.make_async_copy(ones_v, deg_sp.at[idx_v.at[8 * j + q]],
                                  sem_d).wait()
        return carry

    lax.fori_loop(0, NCH_DEG // 8, body, 0)
    plsc.subcore_barrier()

    @pl.when(c == 0)
    def _():
        _copy_node_rows(deg_sp, deg_s, s)

    @pl.when(c == 1)
    def _():
        _copy_node_rows(deg_sp, deg_t, s)


@functools.cache
def _deg_kernel():
    return pl.kernel(
        _deg_body,
        out_type=[jax.ShapeDtypeStruct((NP,), jnp.float32)] * 2,
        mesh=_mesh(),
        scratch_types=[
            pltpu.VMEM((NCH_DEG, K_DEG), jnp.int32),
            pltpu.VMEM((K_DEG,), jnp.float32),
            pltpu.SemaphoreType.DMA,
            pltpu.VMEM_SHARED((NP,), jnp.float32),
        ],
    )


_BM = 1000


def _scale_body(x_ref, w_ref, dgs_ref, dgt_ref, ys0, ys1, yt0, yt1):
    xw = jnp.dot(x_ref[...].astype(jnp.bfloat16),
                 w_ref[...].astype(jnp.bfloat16),
                 preferred_element_type=jnp.float32)
    dis_s = lax.rsqrt(dgs_ref[...] + 1.0)
    dis_t = lax.rsqrt(dgt_ref[...] + 1.0)
    ys = xw[:, :D_OUT] * dis_s
    yt = xw[:, D_OUT:] * dis_t
    ys0[...] = ys[:, :H]
    ys1[...] = ys[:, H:]
    yt0[...] = yt[:, :H]
    yt1[...] = yt[:, H:]


def _scale_matmul(x, w_cat, deg_s, deg_t):
    return pl.pallas_call(
        _scale_body,
        grid=(N // _BM,),
        in_specs=[
            pl.BlockSpec((_BM, D_IN), lambda i: (i, 0)),
            pl.BlockSpec((D_IN, 2 * D_OUT), lambda i: (0, 0)),
            pl.BlockSpec((_BM, 1), lambda i: (i, 0)),
            pl.BlockSpec((_BM, 1), lambda i: (i, 0)),
        ],
        out_specs=[pl.BlockSpec((_BM, H), lambda i: (i, 0))] * 4,
        out_shape=[jax.ShapeDtypeStruct((NP, H), jnp.float32)] * 4,
    )(x, w_cat, deg_s, deg_t)


_NBUF = 3


def _agg_body(y_s0, y_s1, y_t0, y_t1, src_e, dst_e,
              o_s0, o_s1, o_t0, o_t1,
              src_v, win_v, r0, r1, r2, sem_g, sem_s, agg_sp):
    c = lax.axis_index("c")
    s = lax.axis_index("s")
    rows = [r0, r1, r2]

    def gather_chunk(y_ref, i, buf):
        pltpu.async_copy(y_ref.at[src_v.at[pl.ds(i * K_AGG, K_AGG)]],
                         rows[buf], sem_g)

    def wait_gather(y_ref, i, buf):
        pltpu.make_async_copy(y_ref.at[src_v.at[pl.ds(i * K_AGG, K_AGG)]],
                              rows[buf], sem_g).wait()

    def wait_scatter(buf, r):
        pltpu.make_async_copy(rows[buf], agg_sp.at[win_v.at[r]], sem_s).wait()

    def run_graph(g, y_ref, o_ref):
        pltpu.sync_copy(src_e.at[g * NS + s], src_v)
        _copy_node_rows(y_ref, agg_sp, s)
        plsc.subcore_barrier()

        for b in range(_NBUF - 1):
            gather_chunk(y_ref, b, b)

        def window(w, carry):
            @pl.when(w > 0)
            def _():
                wait_scatter((W_AGG - 1) % _NBUF, W_AGG - 1)

            pltpu.sync_copy(dst_e.at[(g * NS + s) * NWIN + w], win_v)
            for r in range(W_AGG):
                i = w * W_AGG + r
                b = r % _NBUF
                wait_gather(y_ref, i, b)
                pltpu.async_copy(rows[b], agg_sp.at[win_v.at[r]], sem_s,
                                 add=True)
                if r >= 1:
                    wait_scatter((r - 1) % _NBUF, r - 1)

                @pl.when(i + _NBUF - 1 < NCH_AGG)
                def _():
                    gather_chunk(y_ref, i + _NBUF - 1, (r + 2) % _NBUF)
            return carry

        lax.fori_loop(0, NWIN, window, 0)
        wait_scatter((W_AGG - 1) % _NBUF, W_AGG - 1)
        plsc.subcore_barrier()
        _copy_node_rows(agg_sp, o_ref, s)
        plsc.subcore_barrier()

    @pl.when(c == 0)
    def _():
        run_graph(0, y_s0, o_s0)
        run_graph(1, y_t0, o_t0)

    @pl.when(c == 1)
    def _():
        run_graph(0, y_s1, o_s1)
        run_graph(1, y_t1, o_t1)


@functools.cache
def _agg_kernel():
    return pl.kernel(
        _agg_body,
        out_type=[jax.ShapeDtypeStruct((NP, H), jnp.float32)] * 4,
        mesh=_mesh(),
        scratch_types=[
            pltpu.VMEM((EPW,), jnp.int32),
            pltpu.VMEM((W_AGG, K_AGG), jnp.int32),
            pltpu.VMEM((K_AGG, H), jnp.float32),
            pltpu.VMEM((K_AGG, H), jnp.float32),
            pltpu.VMEM((K_AGG, H), jnp.float32),
            pltpu.SemaphoreType.DMA,
            pltpu.SemaphoreType.DMA,
            pltpu.VMEM_SHARED((NP, H), jnp.float32),
        ],
    )


def _fuse_body(as0, as1, at0, at1, dgs_ref, dgt_ref, wf_ref, bs_ref, bt_ref,
               bf_ref, out_ref):
    dis_s = lax.rsqrt(dgs_ref[...] + 1.0)
    dis_t = lax.rsqrt(dgt_ref[...] + 1.0)
    hs_l = jnp.maximum(as0[...] * dis_s + bs_ref[:, :H], 0.0).astype(jnp.bfloat16)
    hs_r = jnp.maximum(as1[...] * dis_s + bs_ref[:, H:], 0.0).astype(jnp.bfloat16)
    ht_l = jnp.maximum(at0[...] * dis_t + bt_ref[:, :H], 0.0).astype(jnp.bfloat16)
    ht_r = jnp.maximum(at1[...] * dis_t + bt_ref[:, H:], 0.0).astype(jnp.bfloat16)
    wf = wf_ref[...].astype(jnp.bfloat16)
    acc = bf_ref[...]
    acc = acc + jnp.dot(hs_l, wf[0:H], preferred_element_type=jnp.float32)
    acc = acc + jnp.dot(hs_r, wf[H:2 * H], preferred_element_type=jnp.float32)
    acc = acc + jnp.dot(ht_l, wf[2 * H:3 * H], preferred_element_type=jnp.float32)
    acc = acc + jnp.dot(ht_r, wf[3 * H:4 * H], preferred_element_type=jnp.float32)
    out_ref[...] = jnp.maximum(acc, 0.0)


def _fuse(as0, as1, at0, at1, deg_s, deg_t, w_fuse, b_spa, b_tra, b_fuse):
    return pl.pallas_call(
        _fuse_body,
        grid=(N // _BM,),
        in_specs=[
            pl.BlockSpec((_BM, H), lambda i: (i, 0)),
            pl.BlockSpec((_BM, H), lambda i: (i, 0)),
            pl.BlockSpec((_BM, H), lambda i: (i, 0)),
            pl.BlockSpec((_BM, H), lambda i: (i, 0)),
            pl.BlockSpec((_BM, 1), lambda i: (i, 0)),
            pl.BlockSpec((_BM, 1), lambda i: (i, 0)),
            pl.BlockSpec((2 * D_OUT, D_OUT), lambda i: (0, 0)),
            pl.BlockSpec((1, D_OUT), lambda i: (0, 0)),
            pl.BlockSpec((1, D_OUT), lambda i: (0, 0)),
            pl.BlockSpec((1, D_OUT), lambda i: (0, 0)),
        ],
        out_specs=pl.BlockSpec((_BM, D_OUT), lambda i: (i, 0)),
        out_shape=jax.ShapeDtypeStruct((N, D_OUT), jnp.float32),
    )(as0, as1, at0, at1, deg_s, deg_t, w_fuse, b_spa, b_tra, b_fuse)


def kernel(x, sp_ei, tr_ei, W_spa, b_spa, W_tra, b_tra, W_fuse, b_fuse):
    sp_ei = sp_ei.astype(jnp.int32)
    tr_ei = tr_ei.astype(jnp.int32)

    pad = N + jnp.arange(EP - E, dtype=jnp.int32) % (NP - N)
    dst_s_p = jnp.concatenate([sp_ei[1], pad]).reshape(NS, NCH_DEG, K_DEG)
    dst_t_p = jnp.concatenate([tr_ei[1], pad]).reshape(NS, NCH_DEG, K_DEG)
    ones_h = jnp.ones((K_DEG,), jnp.float32)
    zeros_h = jnp.zeros((RPT,), jnp.float32)
    deg_s, deg_t = _deg_kernel()(dst_s_p, dst_t_p, ones_h, zeros_h)
    deg_s = deg_s.reshape(NP, 1)
    deg_t = deg_t.reshape(NP, 1)

    w_cat = jnp.concatenate([W_spa, W_tra], axis=1)
    ys0, ys1, yt0, yt1 = _scale_matmul(x, w_cat, deg_s, deg_t)

    pad_a = (N + jnp.arange(EPW - E // NS, dtype=jnp.int32) % (NP - N))
    pad_a = jnp.broadcast_to(pad_a, (2 * NS, EPW - E // NS))

    def _pad_edges(a):
        return jnp.concatenate([a.reshape(2 * NS, E // NS), pad_a], axis=1)

    src_e = _pad_edges(jnp.stack([sp_ei[0], tr_ei[0]]))
    dst_e = _pad_edges(jnp.stack([sp_ei[1], tr_ei[1]]))
    dst_e = dst_e.reshape(2 * NS * NWIN, W_AGG, K_AGG)
    os0, os1, ot0, ot1 = _agg_kernel()(ys0, ys1, yt0, yt1, src_e, dst_e)

    return _fuse(os0, os1, ot0, ot1, deg_s, deg_t, W_fuse,
                 b_spa.reshape(1, D_OUT), b_tra.reshape(1, D_OUT),
                 b_fuse.reshape(1, D_OUT))


# speedup vs baseline: 1.0965x; 1.0005x over previous
import functools

import jax
import jax.numpy as jnp
from jax import lax
from jax.experimental import pallas as pl
from jax.experimental.pallas import tpu as pltpu
from jax.experimental.pallas import tpu_sc as plsc

N = 10000
D_IN = 256
D_OUT = 256
H = 128
E = 160000
NS = 16
NP = NS * 640
RPT = 640

K_AGG = 80
NCH_AGG = 126
W_AGG = 42
NWIN = NCH_AGG // W_AGG
EPW = NCH_AGG * K_AGG

K_DEG = 128
NCH_DEG = 80
EP = NS * K_DEG * NCH_DEG


def _mesh():
    return plsc.VectorSubcoreMesh(core_axis_name="c", subcore_axis_name="s")


def _copy_node_rows(src, dst, s):
    pltpu.sync_copy(src.at[pl.ds(s * RPT, RPT)], dst.at[pl.ds(s * RPT, RPT)])


def _deg_body(dst_s, dst_t, ones_h, zeros_h, deg_s, deg_t, idx_v, ones_v,
              sem_d, deg_sp):
    c = lax.axis_index("c")
    s = lax.axis_index("s")
    pltpu.sync_copy(ones_h, ones_v)
    pltpu.sync_copy(zeros_h, deg_sp.at[pl.ds(s * RPT, RPT)])

    @pl.when(c == 0)
    def _():
        pltpu.sync_copy(dst_s.at[s], idx_v)

    @pl.when(c == 1)
    def _():
        pltpu.sync_copy(dst_t.at[s], idx_v)

    plsc.subcore_barrier()

    def body(j, carry):
        for q in range(16):
            pltpu.async_copy(ones_v, deg_sp.at[idx_v.at[16 * j + q]], sem_d,
                             add=True)
        for q in range(16):
            pltpu.make_async_copy(ones_v, deg_sp.at[idx_v.at[16 * j + q]],
                                  sem_d).wait()
        return carry

    lax.fori_loop(0, NCH_DEG // 16, body, 0)
    plsc.subcore_barrier()

    @pl.when(c == 0)
    def _():
        _copy_node_rows(deg_sp, deg_s, s)

    @pl.when(c == 1)
    def _():
        _copy_node_rows(deg_sp, deg_t, s)


@functools.cache
def _deg_kernel():
    return pl.kernel(
        _deg_body,
        out_type=[jax.ShapeDtypeStruct((NP,), jnp.float32)] * 2,
        mesh=_mesh(),
        scratch_types=[
            pltpu.VMEM((NCH_DEG, K_DEG), jnp.int32),
            pltpu.VMEM((K_DEG,), jnp.float32),
            pltpu.SemaphoreType.DMA,
            pltpu.VMEM_SHARED((NP,), jnp.float32),
        ],
    )


_BM = 1000


def _scale_body(x_ref, w_ref, dgs_ref, dgt_ref, ys0, ys1, yt0, yt1):
    xw = jnp.dot(x_ref[...].astype(jnp.bfloat16),
                 w_ref[...].astype(jnp.bfloat16),
                 preferred_element_type=jnp.float32)
    dis_s = lax.rsqrt(dgs_ref[...] + 1.0)
    dis_t = lax.rsqrt(dgt_ref[...] + 1.0)
    ys = xw[:, :D_OUT] * dis_s
    yt = xw[:, D_OUT:] * dis_t
    ys0[...] = ys[:, :H]
    ys1[...] = ys[:, H:]
    yt0[...] = yt[:, :H]
    yt1[...] = yt[:, H:]


def _scale_matmul(x, w_cat, deg_s, deg_t):
    return pl.pallas_call(
        _scale_body,
        grid=(N // _BM,),
        in_specs=[
            pl.BlockSpec((_BM, D_IN), lambda i: (i, 0)),
            pl.BlockSpec((D_IN, 2 * D_OUT), lambda i: (0, 0)),
            pl.BlockSpec((_BM, 1), lambda i: (i, 0)),
            pl.BlockSpec((_BM, 1), lambda i: (i, 0)),
        ],
        out_specs=[pl.BlockSpec((_BM, H), lambda i: (i, 0))] * 4,
        out_shape=[jax.ShapeDtypeStruct((NP, H), jnp.float32)] * 4,
    )(x, w_cat, deg_s, deg_t)


_NBUF = 3


def _agg_body(y_s0, y_s1, y_t0, y_t1, src_e, dst_e,
              o_s0, o_s1, o_t0, o_t1,
              src_v, win_v, r0, r1, r2, sem_g, sem_s, agg_sp):
    c = lax.axis_index("c")
    s = lax.axis_index("s")
    rows = [r0, r1, r2]

    def gather_chunk(y_ref, i, buf):
        pltpu.async_copy(y_ref.at[src_v.at[pl.ds(i * K_AGG, K_AGG)]],
                         rows[buf], sem_g)

    def wait_gather(y_ref, i, buf):
        pltpu.make_async_copy(y_ref.at[src_v.at[pl.ds(i * K_AGG, K_AGG)]],
                              rows[buf], sem_g).wait()

    def wait_scatter(buf, r):
        pltpu.make_async_copy(rows[buf], agg_sp.at[win_v.at[r]], sem_s).wait()

    def run_graph(g, y_ref, o_ref):
        pltpu.sync_copy(src_e.at[g * NS + s], src_v)
        _copy_node_rows(y_ref, agg_sp, s)
        plsc.subcore_barrier()

        for b in range(_NBUF - 1):
            gather_chunk(y_ref, b, b)

        def window(w, carry):
            @pl.when(w > 0)
            def _():
                wait_scatter((W_AGG - 1) % _NBUF, W_AGG - 1)

            pltpu.sync_copy(dst_e.at[(g * NS + s) * NWIN + w], win_v)
            for r in range(W_AGG):
                i = w * W_AGG + r
                b = r % _NBUF
                wait_gather(y_ref, i, b)
                pltpu.async_copy(rows[b], agg_sp.at[win_v.at[r]], sem_s,
                                 add=True)
                if r >= 1:
                    wait_scatter((r - 1) % _NBUF, r - 1)

                @pl.when(i + _NBUF - 1 < NCH_AGG)
                def _():
                    gather_chunk(y_ref, i + _NBUF - 1, (r + 2) % _NBUF)
            return carry

        lax.fori_loop(0, NWIN, window, 0)
        wait_scatter((W_AGG - 1) % _NBUF, W_AGG - 1)
        plsc.subcore_barrier()
        _copy_node_rows(agg_sp, o_ref, s)
        plsc.subcore_barrier()

    @pl.when(c == 0)
    def _():
        run_graph(0, y_s0, o_s0)
        run_graph(1, y_t0, o_t0)

    @pl.when(c == 1)
    def _():
        run_graph(0, y_s1, o_s1)
        run_graph(1, y_t1, o_t1)


@functools.cache
def _agg_kernel():
    return pl.kernel(
        _agg_body,
        out_type=[jax.ShapeDtypeStruct((NP, H), jnp.float32)] * 4,
        mesh=_mesh(),
        scratch_types=[
            pltpu.VMEM((EPW,), jnp.int32),
            pltpu.VMEM((W_AGG, K_AGG), jnp.int32),
            pltpu.VMEM((K_AGG, H), jnp.float32),
            pltpu.VMEM((K_AGG, H), jnp.float32),
            pltpu.VMEM((K_AGG, H), jnp.float32),
            pltpu.SemaphoreType.DMA,
            pltpu.SemaphoreType.DMA,
            pltpu.VMEM_SHARED((NP, H), jnp.float32),
        ],
    )


def _fuse_body(as0, as1, at0, at1, dgs_ref, dgt_ref, wf_ref, bs_ref, bt_ref,
               bf_ref, out_ref):
    dis_s = lax.rsqrt(dgs_ref[...] + 1.0)
    dis_t = lax.rsqrt(dgt_ref[...] + 1.0)
    hs_l = jnp.maximum(as0[...] * dis_s + bs_ref[:, :H], 0.0).astype(jnp.bfloat16)
    hs_r = jnp.maximum(as1[...] * dis_s + bs_ref[:, H:], 0.0).astype(jnp.bfloat16)
    ht_l = jnp.maximum(at0[...] * dis_t + bt_ref[:, :H], 0.0).astype(jnp.bfloat16)
    ht_r = jnp.maximum(at1[...] * dis_t + bt_ref[:, H:], 0.0).astype(jnp.bfloat16)
    wf = wf_ref[...].astype(jnp.bfloat16)
    acc = bf_ref[...]
    acc = acc + jnp.dot(hs_l, wf[0:H], preferred_element_type=jnp.float32)
    acc = acc + jnp.dot(hs_r, wf[H:2 * H], preferred_element_type=jnp.float32)
    acc = acc + jnp.dot(ht_l, wf[2 * H:3 * H], preferred_element_type=jnp.float32)
    acc = acc + jnp.dot(ht_r, wf[3 * H:4 * H], preferred_element_type=jnp.float32)
    out_ref[...] = jnp.maximum(acc, 0.0)


def _fuse(as0, as1, at0, at1, deg_s, deg_t, w_fuse, b_spa, b_tra, b_fuse):
    return pl.pallas_call(
        _fuse_body,
        grid=(N // _BM,),
        in_specs=[
            pl.BlockSpec((_BM, H), lambda i: (i, 0)),
            pl.BlockSpec((_BM, H), lambda i: (i, 0)),
            pl.BlockSpec((_BM, H), lambda i: (i, 0)),
            pl.BlockSpec((_BM, H), lambda i: (i, 0)),
            pl.BlockSpec((_BM, 1), lambda i: (i, 0)),
            pl.BlockSpec((_BM, 1), lambda i: (i, 0)),
            pl.BlockSpec((2 * D_OUT, D_OUT), lambda i: (0, 0)),
            pl.BlockSpec((1, D_OUT), lambda i: (0, 0)),
            pl.BlockSpec((1, D_OUT), lambda i: (0, 0)),
            pl.BlockSpec((1, D_OUT), lambda i: (0, 0)),
        ],
        out_specs=pl.BlockSpec((_BM, D_OUT), lambda i: (i, 0)),
        out_shape=jax.ShapeDtypeStruct((N, D_OUT), jnp.float32),
    )(as0, as1, at0, at1, deg_s, deg_t, w_fuse, b_spa, b_tra, b_fuse)


def kernel(x, sp_ei, tr_ei, W_spa, b_spa, W_tra, b_tra, W_fuse, b_fuse):
    sp_ei = sp_ei.astype(jnp.int32)
    tr_ei = tr_ei.astype(jnp.int32)

    pad = N + jnp.arange(EP - E, dtype=jnp.int32) % (NP - N)
    dst_s_p = jnp.concatenate([sp_ei[1], pad]).reshape(NS, NCH_DEG, K_DEG)
    dst_t_p = jnp.concatenate([tr_ei[1], pad]).reshape(NS, NCH_DEG, K_DEG)
    ones_h = jnp.ones((K_DEG,), jnp.float32)
    zeros_h = jnp.zeros((RPT,), jnp.float32)
    deg_s, deg_t = _deg_kernel()(dst_s_p, dst_t_p, ones_h, zeros_h)
    deg_s = deg_s.reshape(NP, 1)
    deg_t = deg_t.reshape(NP, 1)

    w_cat = jnp.concatenate([W_spa, W_tra], axis=1)
    ys0, ys1, yt0, yt1 = _scale_matmul(x, w_cat, deg_s, deg_t)

    pad_a = (N + jnp.arange(EPW - E // NS, dtype=jnp.int32) % (NP - N))
    pad_a = jnp.broadcast_to(pad_a, (2 * NS, EPW - E // NS))

    def _pad_edges(a):
        return jnp.concatenate([a.reshape(2 * NS, E // NS), pad_a], axis=1)

    src_e = _pad_edges(jnp.stack([sp_ei[0], tr_ei[0]]))
    dst_e = _pad_edges(jnp.stack([sp_ei[1], tr_ei[1]]))
    dst_e = dst_e.reshape(2 * NS * NWIN, W_AGG, K_AGG)
    os0, os1, ot0, ot1 = _agg_kernel()(ys0, ys1, yt0, yt1, src_e, dst_e)

    return _fuse(os0, os1, ot0, ot1, deg_s, deg_t, W_fuse,
                 b_spa.reshape(1, D_OUT), b_tra.reshape(1, D_OUT),
                 b_fuse.reshape(1, D_OUT))
